# baseline (speedup 1.0000x reference)
import functools

import jax
import jax.numpy as jnp
from jax import lax
from jax.experimental import pallas as pl
from jax.experimental.pallas import tpu as pltpu
from jax.experimental.pallas import tpu_sc as plsc

_B, _S, _D = 4, 8192, 1024
_C = 16
_S_SC = 512
_S_TC = _S - _S_SC
_RB = 512


def _make_sc_kernel():
    info = plsc.get_sparse_core_info()
    nc, ns = info.num_cores, info.num_subcores
    nw = nc * ns
    spw = _S_SC // nw
    n_chunks = spw // _C

    mesh = plsc.VectorSubcoreMesh(core_axis_name="c", subcore_axis_name="s")

    @functools.partial(
        pl.kernel,
        mesh=mesh,
        out_type=jax.ShapeDtypeStruct((_B * _S_SC, _D), jnp.float32),
        compiler_params=pltpu.CompilerParams(use_tc_tiling_on_sc=True),
        scratch_types=[
            pltpu.VMEM((_C, _D), jnp.float32),
            pltpu.VMEM((2, _C, _D), jnp.float32),
            pltpu.SemaphoreType.DMA((2,)),
            pltpu.SemaphoreType.DMA((2,)),
        ],
    )
    def sc_add(x_hbm, pos_hbm, out_hbm, pos_v, x_v, isem, osem):
        wid = lax.axis_index("s") * nc + lax.axis_index("c")
        s_base = _S_TC + wid * spw

        items = [(c, b) for c in range(n_chunks) for b in range(_B)]
        n_items = len(items)

        def x_row(i):
            c, b = items[i]
            return b * _S + s_base + c * _C

        def o_row(i):
            c, b = items[i]
            return b * _S_SC + (s_base - _S_TC) + c * _C

        def start_in(i):
            pltpu.async_copy(x_hbm.at[pl.ds(x_row(i), _C), :],
                             x_v.at[i % 2], isem.at[i % 2])

        def wait_in(i):
            pltpu.make_async_copy(x_hbm.at[pl.ds(x_row(i), _C), :],
                                  x_v.at[i % 2], isem.at[i % 2]).wait()

        def start_out(i):
            pltpu.async_copy(x_v.at[i % 2],
                             out_hbm.at[pl.ds(o_row(i), _C), :],
                             osem.at[i % 2])

        def wait_out(i):
            pltpu.make_async_copy(x_v.at[i % 2],
                                  out_hbm.at[pl.ds(o_row(i), _C), :],
                                  osem.at[i % 2]).wait()

        start_in(0)
        for i, (c, b) in enumerate(items):
            if i + 1 < n_items:
                if i >= 1:
                    wait_out(i - 1)
                start_in(i + 1)
            if b == 0:
                pltpu.sync_copy(
                    pos_hbm.at[pl.ds(s_base + c * _C, _C), :], pos_v)
            wait_in(i)

            xbuf = x_v.at[i % 2]
            for r in range(_C):
                @plsc.parallel_loop(0, _D, 16, unroll=8)
                def _(j):
                    plsc.addupdate(xbuf.at[r, pl.ds(j, 16)],
                                   pos_v[r, pl.ds(j, 16)])

            start_out(i)
        wait_out(n_items - 2)
        wait_out(n_items - 1)

    return sc_add


_sc_add = _make_sc_kernel()


def _tc_body(x_ref, p_ref, o_ref):
    o_ref[...] = x_ref[...] + p_ref[...]


def kernel(x, pos_table):
    b, s, d = x.shape
    nsb = _S_TC // _RB
    tc_out = pl.pallas_call(
        _tc_body,
        grid=(nsb, b),
        in_specs=[
            pl.BlockSpec((1, _RB, d), lambda i, j: (j, i, 0)),
            pl.BlockSpec((_RB, d), lambda i, j: (i, 0)),
        ],
        out_specs=pl.BlockSpec((1, _RB, d), lambda i, j: (j, i, 0)),
        out_shape=jax.ShapeDtypeStruct((b, s, d), x.dtype),
    )(x, pos_table)
    sc_out = _sc_add(x.reshape(b * s, d), pos_table)
    return lax.dynamic_update_slice(
        tc_out, sc_out.reshape(b, _S_SC, d), (0, _S_TC, 0))


# speedup vs baseline: 1.1319x; 1.1319x over previous
import functools

import jax
import jax.numpy as jnp
from jax import lax
from jax.experimental import pallas as pl
from jax.experimental.pallas import tpu as pltpu
from jax.experimental.pallas import tpu_sc as plsc

_B, _S, _D = 4, 8192, 1024
_C = 16
_S_SC = 512
_S_TC = _S - _S_SC
_RB = 1024


def _make_sc_kernel():
    info = plsc.get_sparse_core_info()
    nc, ns = info.num_cores, info.num_subcores
    nw = nc * ns
    spw = _S_SC // nw
    n_chunks = spw // _C

    mesh = plsc.VectorSubcoreMesh(core_axis_name="c", subcore_axis_name="s")

    @functools.partial(
        pl.kernel,
        mesh=mesh,
        out_type=jax.ShapeDtypeStruct((_B * _S_SC, _D), jnp.float32),
        compiler_params=pltpu.CompilerParams(use_tc_tiling_on_sc=True),
        scratch_types=[
            pltpu.VMEM((_C, _D), jnp.float32),
            pltpu.VMEM((2, _C, _D), jnp.float32),
            pltpu.SemaphoreType.DMA((2,)),
            pltpu.SemaphoreType.DMA((2,)),
        ],
    )
    def sc_add(x_hbm, pos_hbm, out_hbm, pos_v, x_v, isem, osem):
        wid = lax.axis_index("s") * nc + lax.axis_index("c")
        s_base = _S_TC + wid * spw

        items = [(c, b) for c in range(n_chunks) for b in range(_B)]
        n_items = len(items)

        def x_row(i):
            c, b = items[i]
            return b * _S + s_base + c * _C

        def o_row(i):
            c, b = items[i]
            return b * _S_SC + (s_base - _S_TC) + c * _C

        def start_in(i):
            pltpu.async_copy(x_hbm.at[pl.ds(x_row(i), _C), :],
                             x_v.at[i % 2], isem.at[i % 2])

        def wait_in(i):
            pltpu.make_async_copy(x_hbm.at[pl.ds(x_row(i), _C), :],
                                  x_v.at[i % 2], isem.at[i % 2]).wait()

        def start_out(i):
            pltpu.async_copy(x_v.at[i % 2],
                             out_hbm.at[pl.ds(o_row(i), _C), :],
                             osem.at[i % 2])

        def wait_out(i):
            pltpu.make_async_copy(x_v.at[i % 2],
                                  out_hbm.at[pl.ds(o_row(i), _C), :],
                                  osem.at[i % 2]).wait()

        start_in(0)
        for i, (c, b) in enumerate(items):
            if i + 1 < n_items:
                if i >= 1:
                    wait_out(i - 1)
                start_in(i + 1)
            if b == 0:
                pltpu.sync_copy(
                    pos_hbm.at[pl.ds(s_base + c * _C, _C), :], pos_v)
            wait_in(i)

            xbuf = x_v.at[i % 2]
            for r in range(_C):
                @plsc.parallel_loop(0, _D, 16, unroll=8)
                def _(j):
                    plsc.addupdate(xbuf.at[r, pl.ds(j, 16)],
                                   pos_v[r, pl.ds(j, 16)])

            start_out(i)
        wait_out(n_items - 2)
        wait_out(n_items - 1)

    return sc_add


_sc_add = _make_sc_kernel()


def _tc_body(x_ref, p_ref, o_ref):
    o_ref[...] = x_ref[...] + p_ref[...]


def kernel(x, pos_table):
    b, s, d = x.shape
    nsb = _S_TC // _RB
    tc_out = pl.pallas_call(
        _tc_body,
        grid=(nsb, b),
        in_specs=[
            pl.BlockSpec((1, _RB, d), lambda i, j: (j, i, 0)),
            pl.BlockSpec((_RB, d), lambda i, j: (i, 0)),
        ],
        out_specs=pl.BlockSpec((1, _RB, d), lambda i, j: (j, i, 0)),
        out_shape=jax.ShapeDtypeStruct((b, s, d), x.dtype),
    )(x, pos_table)
    sc_out = _sc_add(x.reshape(b * s, d), pos_table)
    return lax.dynamic_update_slice(
        tc_out, sc_out.reshape(b, _S_SC, d), (0, _S_TC, 0))
